# baseline (speedup 1.0000x reference)
import jax
import jax.numpy as jnp
from jax import lax
from jax.experimental import pallas as pl
from jax.experimental.pallas import tpu as pltpu
from jax.experimental.pallas import tpu_sc as plsc

L_DIM = 100000
EMBED_DIM = 32
R = L_DIM - 1
NC = 2
NS = 16
NW = NC * NS
LANES = 16

TCW = 128
SPAN = 25
NB = SPAN
NBUF = 4
UNIFORM_TC = 780
LAST_F = UNIFORM_TC - SPAN
TAIL0 = UNIFORM_TC * TCW
KPV = TCW // LANES

CC = 3128
CC_LAST = R - (NW - 1) * CC
CBUF = 3136


def _sc_body(wu, wv, out_u, out_v, out_c, biu, bou, biv, bov, buf_c,
             s_ui, s_vi, s_uo, s_vo, s_c):
    wid = lax.axis_index("s") * NC + lax.axis_index("c")
    iota16 = lax.iota(jnp.int32, LANES)
    perm = jnp.where(iota16 < LANES - 1, iota16 + 1, 0)
    low15 = iota16 < LANES - 1

    def rot(v):
        return v.at[perm].get(mode="promise_in_bounds", unique_indices=True)

    def elu1(y):
        return jnp.where(y > 0.0, y + 1.0, jnp.exp(y))

    cbase = wid * CC + 1

    def c_body(j, carry):
        buf_c[pl.ds(j * LANES, LANES)] = cbase + j * LANES + iota16
        return carry

    lax.fori_loop(0, CBUF // LANES, c_body, 0, unroll=4)

    @pl.when(wid < NW - 1)
    def _():
        pltpu.async_copy(buf_c.at[pl.ds(0, CC)],
                         out_c.at[pl.ds(wid * CC, CC)], s_c).wait()

    @pl.when(wid == NW - 1)
    def _():
        pltpu.async_copy(buf_c.at[pl.ds(0, CC_LAST)],
                         out_c.at[pl.ds((NW - 1) * CC, CC_LAST)], s_c).wait()

    col0 = (wid * LAST_F) // (NW - 1) * TCW

    def shift_block(src_u, src_v, la_u, la_v, dst_u, dst_v):
        def d_body(d, carry):
            ru = rot(src_u[d, pl.ds(0, LANES)])
            rv = rot(src_v[d, pl.ds(0, LANES)])
            for k in range(KPV):
                if k < KPV - 1:
                    nu = src_u[d, pl.ds((k + 1) * LANES, LANES)]
                    nv = src_v[d, pl.ds((k + 1) * LANES, LANES)]
                else:
                    nu = la_u[d, pl.ds(0, LANES)]
                    nv = la_v[d, pl.ds(0, LANES)]
                ru_b, rv_b = rot(nu), rot(nv)
                dst_u[d, pl.ds(k * LANES, LANES)] = jnp.where(low15, ru, ru_b)
                dst_v[d, pl.ds(k * LANES, LANES)] = elu1(
                    jnp.where(low15, rv, rv_b))
                ru, rv = ru_b, rv_b
            return carry

        lax.fori_loop(0, EMBED_DIM, d_body, 0, unroll=2)

    in_copies = {}

    def issue_in(b):
        sl = b % NBUF
        t = col0 + b * TCW
        in_copies[b] = (
            pltpu.async_copy(wu.at[:, pl.ds(t, TCW)], biu[sl], s_ui[sl]),
            pltpu.async_copy(wv.at[:, pl.ds(t, TCW)], biv[sl], s_vi[sl]),
        )

    out_copies = {}
    for b in range(NBUF):
        issue_in(b)
    cu, cv = in_copies.pop(0)
    cu.wait()
    cv.wait()

    for b in range(NB):
        sl = b % NBUF
        la = (b + 1) % NBUF
        cu, cv = in_copies.pop(b + 1)
        cu.wait()
        cv.wait()
        if b - NBUF >= 0:
            pu, pv = out_copies.pop(b - NBUF)
            pu.wait()
            pv.wait()
        shift_block(biu[sl], biv[sl], biu[la], biv[la], bou[sl], bov[sl])
        t = col0 + b * TCW
        out_copies[b] = (
            pltpu.async_copy(bou[sl], out_u.at[:, pl.ds(t, TCW)], s_uo[sl]),
            pltpu.async_copy(bov[sl], out_v.at[:, pl.ds(t, TCW)], s_vo[sl]),
        )
        if b + NBUF <= NB:
            issue_in(b + NBUF)

    for b in sorted(out_copies):
        pu, pv = out_copies[b]
        pu.wait()
        pv.wait()


_sc_lookup = pl.kernel(
    _sc_body,
    out_type=(
        jax.ShapeDtypeStruct((EMBED_DIM, R), jnp.float32),
        jax.ShapeDtypeStruct((EMBED_DIM, R), jnp.float32),
        jax.ShapeDtypeStruct((R,), jnp.int32),
    ),
    mesh=plsc.VectorSubcoreMesh(core_axis_name="c", subcore_axis_name="s",
                                num_cores=NC, num_subcores=NS),
    compiler_params=pltpu.CompilerParams(use_tc_tiling_on_sc=True),
    scratch_types=[
        [pltpu.VMEM((EMBED_DIM, TCW), jnp.float32)] * NBUF,
        [pltpu.VMEM((EMBED_DIM, TCW), jnp.float32)] * NBUF,
        [pltpu.VMEM((EMBED_DIM, TCW), jnp.float32)] * NBUF,
        [pltpu.VMEM((EMBED_DIM, TCW), jnp.float32)] * NBUF,
        pltpu.VMEM((CBUF,), jnp.int32),
        [pltpu.SemaphoreType.DMA] * NBUF,
        [pltpu.SemaphoreType.DMA] * NBUF,
        [pltpu.SemaphoreType.DMA] * NBUF,
        [pltpu.SemaphoreType.DMA] * NBUF,
        pltpu.SemaphoreType.DMA,
    ],
)

_FIXW = 256
_FIXB = TAIL0 // _FIXW
_TIN = L_DIM - TAIL0


def _tc_fix_body(wu_ref, wv_ref, scu_ref, scv_ref, ou_ref, ov_ref):
    del scu_ref, scv_ref
    pad = jnp.zeros((EMBED_DIM, _FIXW - (_TIN - 1)), jnp.float32)
    su = jnp.concatenate([wu_ref[:, 1:], pad], axis=1)
    sv = jnp.concatenate([wv_ref[:, 1:], pad], axis=1)
    ou_ref[...] = su
    ov_ref[...] = jnp.where(sv > 0.0, sv + 1.0, jnp.exp(sv))


_tc_fix = pl.pallas_call(
    _tc_fix_body,
    grid=(1,),
    in_specs=[
        pl.BlockSpec((EMBED_DIM, _TIN), lambda i: (0, 0)),
        pl.BlockSpec((EMBED_DIM, _TIN), lambda i: (0, 0)),
        pl.BlockSpec((8, TCW), lambda i: (0, 0)),
        pl.BlockSpec((8, TCW), lambda i: (0, 0)),
    ],
    out_specs=[
        pl.BlockSpec((EMBED_DIM, _FIXW), lambda i: (0, _FIXB)),
        pl.BlockSpec((EMBED_DIM, _FIXW), lambda i: (0, _FIXB)),
    ],
    out_shape=[
        jax.ShapeDtypeStruct((EMBED_DIM, R), jnp.float32),
        jax.ShapeDtypeStruct((EMBED_DIM, R), jnp.float32),
    ],
    input_output_aliases={2: 0, 3: 1},
)


def kernel(traj, traj_len, W_u, W_v):
    del traj, traj_len
    wut = jnp.swapaxes(W_u, 0, 1)
    wvt = jnp.swapaxes(W_v, 0, 1)
    scu, scv, candidates = _sc_lookup(wut, wvt)
    wu_tail = lax.slice(wut, (0, TAIL0), (EMBED_DIM, L_DIM))
    wv_tail = lax.slice(wvt, (0, TAIL0), (EMBED_DIM, L_DIM))
    out_u_t, out_v_t = _tc_fix(wu_tail, wv_tail, scu, scv)
    return (jnp.swapaxes(out_u_t, 0, 1), jnp.swapaxes(out_v_t, 0, 1),
            candidates)


# speedup vs baseline: 2.3730x; 2.3730x over previous
import jax
import jax.numpy as jnp
from jax import lax
from jax.experimental import pallas as pl
from jax.experimental.pallas import tpu as pltpu
from jax.experimental.pallas import tpu_sc as plsc

L_DIM = 100000
EMBED_DIM = 32
R = L_DIM - 1
NC = 2
NS = 16
NW = NC * NS
LANES = 16

TCW = 128
SPAN = 25
NB = SPAN
NBUF = 4
UNIFORM_TC = 780
LAST_F = UNIFORM_TC - SPAN
TAIL0 = UNIFORM_TC * TCW
KPV = TCW // LANES

CC = 3128
CC_LAST = R - (NW - 1) * CC
CBUF = 3136


def _sc_body(wu, wv, out_u, out_v, out_c, biu, bou, biv, bov, buf_c,
             s_ui, s_vi, s_uo, s_vo, s_c):
    wid = lax.axis_index("s") * NC + lax.axis_index("c")
    iota16 = lax.iota(jnp.int32, LANES)
    perm = jnp.where(iota16 < LANES - 1, iota16 + 1, 0)
    low15 = iota16 < LANES - 1

    def rot(v):
        return v.at[perm].get(mode="promise_in_bounds", unique_indices=True)

    def elu1(y):
        return jnp.where(y > 0.0, y + 1.0, jnp.exp(y))

    cbase = wid * CC + 1

    def c_body(j, carry):
        buf_c[pl.ds(j * LANES, LANES)] = cbase + j * LANES + iota16
        return carry

    lax.fori_loop(0, CBUF // LANES, c_body, 0, unroll=4)

    @pl.when(wid < NW - 1)
    def _():
        pltpu.async_copy(buf_c.at[pl.ds(0, CC)],
                         out_c.at[pl.ds(wid * CC, CC)], s_c).wait()

    @pl.when(wid == NW - 1)
    def _():
        pltpu.async_copy(buf_c.at[pl.ds(0, CC_LAST)],
                         out_c.at[pl.ds((NW - 1) * CC, CC_LAST)], s_c).wait()

    col0 = (wid * LAST_F) // (NW - 1) * TCW

    def shift_block(src_u, src_v, la_u, la_v, dst_u, dst_v):
        def d_body(d, carry):
            ru = rot(src_u[d, pl.ds(0, LANES)])
            rv = rot(src_v[d, pl.ds(0, LANES)])
            for k in range(KPV):
                if k < KPV - 1:
                    nu = src_u[d, pl.ds((k + 1) * LANES, LANES)]
                    nv = src_v[d, pl.ds((k + 1) * LANES, LANES)]
                else:
                    nu = la_u[d, pl.ds(0, LANES)]
                    nv = la_v[d, pl.ds(0, LANES)]
                ru_b, rv_b = rot(nu), rot(nv)
                dst_u[d, pl.ds(k * LANES, LANES)] = jnp.where(low15, ru, ru_b)
                dst_v[d, pl.ds(k * LANES, LANES)] = elu1(
                    jnp.where(low15, rv, rv_b))
                ru, rv = ru_b, rv_b
            return carry

        lax.fori_loop(0, EMBED_DIM, d_body, 0)

    in_copies = {}

    def issue_in(b):
        sl = b % NBUF
        t = col0 + b * TCW
        in_copies[b] = (
            pltpu.async_copy(wu.at[:, pl.ds(t, TCW)], biu[sl], s_ui[sl]),
            pltpu.async_copy(wv.at[:, pl.ds(t, TCW)], biv[sl], s_vi[sl]),
        )

    out_copies = {}
    for b in range(NBUF):
        issue_in(b)
    cu, cv = in_copies.pop(0)
    cu.wait()
    cv.wait()

    for b in range(NB):
        sl = b % NBUF
        la = (b + 1) % NBUF
        cu, cv = in_copies.pop(b + 1)
        cu.wait()
        cv.wait()
        if b - NBUF >= 0:
            pu, pv = out_copies.pop(b - NBUF)
            pu.wait()
            pv.wait()
        shift_block(biu[sl], biv[sl], biu[la], biv[la], bou[sl], bov[sl])
        t = col0 + b * TCW
        out_copies[b] = (
            pltpu.async_copy(bou[sl], out_u.at[:, pl.ds(t, TCW)], s_uo[sl]),
            pltpu.async_copy(bov[sl], out_v.at[:, pl.ds(t, TCW)], s_vo[sl]),
        )
        if b + NBUF <= NB:
            issue_in(b + NBUF)

    for b in sorted(out_copies):
        pu, pv = out_copies[b]
        pu.wait()
        pv.wait()


_sc_lookup = pl.kernel(
    _sc_body,
    out_type=(
        jax.ShapeDtypeStruct((EMBED_DIM, R), jnp.float32),
        jax.ShapeDtypeStruct((EMBED_DIM, R), jnp.float32),
        jax.ShapeDtypeStruct((R,), jnp.int32),
    ),
    mesh=plsc.VectorSubcoreMesh(core_axis_name="c", subcore_axis_name="s",
                                num_cores=NC, num_subcores=NS),
    compiler_params=pltpu.CompilerParams(use_tc_tiling_on_sc=True),
    scratch_types=[
        [pltpu.VMEM((EMBED_DIM, TCW), jnp.float32)] * NBUF,
        [pltpu.VMEM((EMBED_DIM, TCW), jnp.float32)] * NBUF,
        [pltpu.VMEM((EMBED_DIM, TCW), jnp.float32)] * NBUF,
        [pltpu.VMEM((EMBED_DIM, TCW), jnp.float32)] * NBUF,
        pltpu.VMEM((CBUF,), jnp.int32),
        [pltpu.SemaphoreType.DMA] * NBUF,
        [pltpu.SemaphoreType.DMA] * NBUF,
        [pltpu.SemaphoreType.DMA] * NBUF,
        [pltpu.SemaphoreType.DMA] * NBUF,
        pltpu.SemaphoreType.DMA,
    ],
)

_FIXW = 256
_FIXB = TAIL0 // _FIXW
_TIN = L_DIM - TAIL0


def _tc_fix_body(wu_ref, wv_ref, scu_ref, scv_ref, ou_ref, ov_ref):
    del scu_ref, scv_ref
    pad = jnp.zeros((EMBED_DIM, _FIXW - (_TIN - 1)), jnp.float32)
    su = jnp.concatenate([wu_ref[:, 1:], pad], axis=1)
    sv = jnp.concatenate([wv_ref[:, 1:], pad], axis=1)
    ou_ref[...] = su
    ov_ref[...] = jnp.where(sv > 0.0, sv + 1.0, jnp.exp(sv))


_tc_fix = pl.pallas_call(
    _tc_fix_body,
    grid=(1,),
    in_specs=[
        pl.BlockSpec((EMBED_DIM, _TIN), lambda i: (0, 0)),
        pl.BlockSpec((EMBED_DIM, _TIN), lambda i: (0, 0)),
        pl.BlockSpec((8, TCW), lambda i: (0, 0)),
        pl.BlockSpec((8, TCW), lambda i: (0, 0)),
    ],
    out_specs=[
        pl.BlockSpec((EMBED_DIM, _FIXW), lambda i: (0, _FIXB)),
        pl.BlockSpec((EMBED_DIM, _FIXW), lambda i: (0, _FIXB)),
    ],
    out_shape=[
        jax.ShapeDtypeStruct((EMBED_DIM, R), jnp.float32),
        jax.ShapeDtypeStruct((EMBED_DIM, R), jnp.float32),
    ],
    input_output_aliases={2: 0, 3: 1},
)


def kernel(traj, traj_len, W_u, W_v):
    del traj, traj_len
    wut = jnp.swapaxes(W_u, 0, 1)
    wvt = jnp.swapaxes(W_v, 0, 1)
    scu, scv, candidates = _sc_lookup(wut, wvt)
    wu_tail = lax.slice(wut, (0, TAIL0), (EMBED_DIM, L_DIM))
    wv_tail = lax.slice(wvt, (0, TAIL0), (EMBED_DIM, L_DIM))
    out_u_t, out_v_t = _tc_fix(wu_tail, wv_tail, scu, scv)
    return (jnp.swapaxes(out_u_t, 0, 1), jnp.swapaxes(out_v_t, 0, 1),
            candidates)


# speedup vs baseline: 2.4625x; 1.0377x over previous
import jax
import jax.numpy as jnp
from jax import lax
from jax.experimental import pallas as pl
from jax.experimental.pallas import tpu as pltpu
from jax.experimental.pallas import tpu_sc as plsc

L_DIM = 100000
EMBED_DIM = 32
R = L_DIM - 1
NC = 2
NS = 16
NW = NC * NS
LANES = 16

TCW = 128
SPAN = 25
NB = SPAN
NBUF = 6
UNIFORM_TC = 780
LAST_F = UNIFORM_TC - SPAN
TAIL0 = UNIFORM_TC * TCW
KPV = TCW // LANES

CC = 3128
CC_LAST = R - (NW - 1) * CC
CBUF = 3136


def _sc_body(wu, wv, out_u, out_v, out_c, biu, bou, biv, bov, buf_c,
             s_ui, s_vi, s_uo, s_vo, s_c):
    wid = lax.axis_index("s") * NC + lax.axis_index("c")
    iota16 = lax.iota(jnp.int32, LANES)
    perm = jnp.where(iota16 < LANES - 1, iota16 + 1, 0)
    low15 = iota16 < LANES - 1

    def rot(v):
        return v.at[perm].get(mode="promise_in_bounds", unique_indices=True)

    def elu1(y):
        return jnp.where(y > 0.0, y + 1.0, jnp.exp(y))

    cbase = wid * CC + 1

    def c_body(j, carry):
        buf_c[pl.ds(j * LANES, LANES)] = cbase + j * LANES + iota16
        return carry

    lax.fori_loop(0, CBUF // LANES, c_body, 0, unroll=4)

    @pl.when(wid < NW - 1)
    def _():
        pltpu.async_copy(buf_c.at[pl.ds(0, CC)],
                         out_c.at[pl.ds(wid * CC, CC)], s_c).wait()

    @pl.when(wid == NW - 1)
    def _():
        pltpu.async_copy(buf_c.at[pl.ds(0, CC_LAST)],
                         out_c.at[pl.ds((NW - 1) * CC, CC_LAST)], s_c).wait()

    col0 = (wid * LAST_F) // (NW - 1) * TCW

    def shift_block(src_u, src_v, la_u, la_v, dst_u, dst_v):
        def d_body(d, carry):
            ru = rot(src_u[d, pl.ds(0, LANES)])
            rv = rot(src_v[d, pl.ds(0, LANES)])
            for k in range(KPV):
                if k < KPV - 1:
                    nu = src_u[d, pl.ds((k + 1) * LANES, LANES)]
                    nv = src_v[d, pl.ds((k + 1) * LANES, LANES)]
                else:
                    nu = la_u[d, pl.ds(0, LANES)]
                    nv = la_v[d, pl.ds(0, LANES)]
                ru_b, rv_b = rot(nu), rot(nv)
                dst_u[d, pl.ds(k * LANES, LANES)] = jnp.where(low15, ru, ru_b)
                dst_v[d, pl.ds(k * LANES, LANES)] = elu1(
                    jnp.where(low15, rv, rv_b))
                ru, rv = ru_b, rv_b
            return carry

        lax.fori_loop(0, EMBED_DIM, d_body, 0)

    in_copies = {}

    def issue_in(b):
        sl = b % NBUF
        t = col0 + b * TCW
        in_copies[b] = (
            pltpu.async_copy(wu.at[:, pl.ds(t, TCW)], biu[sl], s_ui[sl]),
            pltpu.async_copy(wv.at[:, pl.ds(t, TCW)], biv[sl], s_vi[sl]),
        )

    out_copies = {}
    for b in range(NBUF):
        issue_in(b)
    cu, cv = in_copies.pop(0)
    cu.wait()
    cv.wait()

    for b in range(NB):
        sl = b % NBUF
        la = (b + 1) % NBUF
        cu, cv = in_copies.pop(b + 1)
        cu.wait()
        cv.wait()
        if b - NBUF >= 0:
            pu, pv = out_copies.pop(b - NBUF)
            pu.wait()
            pv.wait()
        shift_block(biu[sl], biv[sl], biu[la], biv[la], bou[sl], bov[sl])
        t = col0 + b * TCW
        out_copies[b] = (
            pltpu.async_copy(bou[sl], out_u.at[:, pl.ds(t, TCW)], s_uo[sl]),
            pltpu.async_copy(bov[sl], out_v.at[:, pl.ds(t, TCW)], s_vo[sl]),
        )
        if b + NBUF <= NB:
            issue_in(b + NBUF)

    for b in sorted(out_copies):
        pu, pv = out_copies[b]
        pu.wait()
        pv.wait()


_sc_lookup = pl.kernel(
    _sc_body,
    out_type=(
        jax.ShapeDtypeStruct((EMBED_DIM, R), jnp.float32),
        jax.ShapeDtypeStruct((EMBED_DIM, R), jnp.float32),
        jax.ShapeDtypeStruct((R,), jnp.int32),
    ),
    mesh=plsc.VectorSubcoreMesh(core_axis_name="c", subcore_axis_name="s",
                                num_cores=NC, num_subcores=NS),
    compiler_params=pltpu.CompilerParams(use_tc_tiling_on_sc=True),
    scratch_types=[
        [pltpu.VMEM((EMBED_DIM, TCW), jnp.float32)] * NBUF,
        [pltpu.VMEM((EMBED_DIM, TCW), jnp.float32)] * NBUF,
        [pltpu.VMEM((EMBED_DIM, TCW), jnp.float32)] * NBUF,
        [pltpu.VMEM((EMBED_DIM, TCW), jnp.float32)] * NBUF,
        pltpu.VMEM((CBUF,), jnp.int32),
        [pltpu.SemaphoreType.DMA] * NBUF,
        [pltpu.SemaphoreType.DMA] * NBUF,
        [pltpu.SemaphoreType.DMA] * NBUF,
        [pltpu.SemaphoreType.DMA] * NBUF,
        pltpu.SemaphoreType.DMA,
    ],
)

_FIXW = 256
_FIXB = TAIL0 // _FIXW
_TIN = L_DIM - TAIL0


def _tc_fix_body(wu_ref, wv_ref, scu_ref, scv_ref, ou_ref, ov_ref):
    del scu_ref, scv_ref
    pad = jnp.zeros((EMBED_DIM, _FIXW - (_TIN - 1)), jnp.float32)
    su = jnp.concatenate([wu_ref[:, 1:], pad], axis=1)
    sv = jnp.concatenate([wv_ref[:, 1:], pad], axis=1)
    ou_ref[...] = su
    ov_ref[...] = jnp.where(sv > 0.0, sv + 1.0, jnp.exp(sv))


_tc_fix = pl.pallas_call(
    _tc_fix_body,
    grid=(1,),
    in_specs=[
        pl.BlockSpec((EMBED_DIM, _TIN), lambda i: (0, 0)),
        pl.BlockSpec((EMBED_DIM, _TIN), lambda i: (0, 0)),
        pl.BlockSpec((8, TCW), lambda i: (0, 0)),
        pl.BlockSpec((8, TCW), lambda i: (0, 0)),
    ],
    out_specs=[
        pl.BlockSpec((EMBED_DIM, _FIXW), lambda i: (0, _FIXB)),
        pl.BlockSpec((EMBED_DIM, _FIXW), lambda i: (0, _FIXB)),
    ],
    out_shape=[
        jax.ShapeDtypeStruct((EMBED_DIM, R), jnp.float32),
        jax.ShapeDtypeStruct((EMBED_DIM, R), jnp.float32),
    ],
    input_output_aliases={2: 0, 3: 1},
)


def kernel(traj, traj_len, W_u, W_v):
    del traj, traj_len
    wut = jnp.swapaxes(W_u, 0, 1)
    wvt = jnp.swapaxes(W_v, 0, 1)
    scu, scv, candidates = _sc_lookup(wut, wvt)
    wu_tail = lax.slice(wut, (0, TAIL0), (EMBED_DIM, L_DIM))
    wv_tail = lax.slice(wvt, (0, TAIL0), (EMBED_DIM, L_DIM))
    out_u_t, out_v_t = _tc_fix(wu_tail, wv_tail, scu, scv)
    return (jnp.swapaxes(out_u_t, 0, 1), jnp.swapaxes(out_v_t, 0, 1),
            candidates)
